# Initial kernel scaffold; baseline (speedup 1.0000x reference)
import jax
import jax.numpy as jnp
from jax.experimental import pallas as pl


def kernel(x, table, gamma, beta):
    raise NotImplementedError("write your pallas kernel here")


# speedup vs baseline: 2.1296x; 2.1296x over previous
import functools

import jax
import jax.numpy as jnp
from jax import lax
from jax.experimental import pallas as pl
from jax.experimental.pallas import tpu as pltpu
from jax.experimental.pallas import tpu_sc as plsc

NUM_ROWS = 100000
C = 32
L = 16
NC, NS = 2, 16
NW = NC * NS
BLK = 128
EPS = 1e-5


def _rsqrt(v):
    xi = plsc.bitcast(v, jnp.int32)
    yi = jnp.int32(0x5F3759DF) - (xi >> 1)
    y = plsc.bitcast(yi, jnp.float32)
    for _ in range(3):
        y = y * (1.5 - 0.5 * v * y * y)
    return y


def _ln_block(rows_v, g_v, b_v):
    it = jax.lax.iota(jnp.int32, L)
    g_vec = [g_v[pl.ds(0, L)], g_v[pl.ds(L, L)]]
    b_vec = [b_v[pl.ds(0, L)], b_v[pl.ds(L, L)]]
    for g in range(BLK // L):
        row_ids = it + (g * L)
        s = jnp.zeros((L,), jnp.float32)
        q = jnp.zeros((L,), jnp.float32)
        cols = []
        for c in range(C):
            cid = jnp.full((L,), c, jnp.int32)
            xc = plsc.load_gather(rows_v, [row_ids, cid])
            cols.append(xc)
            s = s + xc
            q = q + xc * xc
        mean = s * (1.0 / C)
        var = q * (1.0 / C) - mean * mean
        rstd = _rsqrt(jnp.maximum(var, 0.0) + EPS)
        for c in range(C):
            cid = jnp.full((L,), c, jnp.int32)
            t = (cols[c] - mean) * rstd
            t = t * g_vec[c // L][c % L] + b_vec[c // L][c % L]
            plsc.store_scatter(rows_v, [row_ids, cid], t)


def _make_sc_kernel(n):
    assert n % (NW * BLK) == 0
    blocks_per_w = n // (NW * BLK)
    mesh = plsc.VectorSubcoreMesh(core_axis_name="c", subcore_axis_name="s")

    @functools.partial(
        pl.kernel,
        out_type=jax.ShapeDtypeStruct((n, C), jnp.float32),
        mesh=mesh,
        compiler_params=pltpu.CompilerParams(
            needs_layout_passes=False, use_tc_tiling_on_sc=False),
        scratch_types=[
            pltpu.VMEM((BLK,), jnp.int32),
            pltpu.VMEM((BLK, C), jnp.float32),
            pltpu.VMEM((C,), jnp.float32),
            pltpu.VMEM((C,), jnp.float32),
            pltpu.SemaphoreType.DMA,
        ],
    )
    def sc_kernel(x_hbm, table_hbm, gamma_hbm, beta_hbm, out_hbm,
                  idx_v, rows_v, g_v, b_v, sem):
        wid = lax.axis_index("s") * NC + lax.axis_index("c")
        base = wid * (blocks_per_w * BLK)
        pltpu.sync_copy(gamma_hbm, g_v)
        pltpu.sync_copy(beta_hbm, b_v)
        def body(i, carry):
            row0 = base + i * BLK
            pltpu.sync_copy(x_hbm.at[pl.ds(row0, BLK)], idx_v)
            pltpu.async_copy(table_hbm.at[idx_v], rows_v, sem).wait()
            _ln_block(rows_v, g_v, b_v)
            pltpu.sync_copy(rows_v, out_hbm.at[pl.ds(row0, BLK)])
            return carry

        lax.fori_loop(0, blocks_per_w, body, 0)

    return sc_kernel


def kernel(x, table, gamma, beta):
    b, l = x.shape
    n = b * l
    xf = x.reshape(n).astype(jnp.int32)
    out = _make_sc_kernel(n)(xf, table, gamma, beta)
    return out.reshape(b, l, C)


# speedup vs baseline: 2.7097x; 1.2724x over previous
import functools

import jax
import jax.numpy as jnp
from jax import lax
from jax.experimental import pallas as pl
from jax.experimental.pallas import tpu as pltpu
from jax.experimental.pallas import tpu_sc as plsc

C = 32
L = 16
NC, NS = 2, 16
NW = NC * NS
BLK = 128
EPS = 1e-5


def _rsqrt(v):
    xi = plsc.bitcast(v, jnp.int32)
    yi = jnp.int32(0x5F3759DF) - (xi >> 1)
    y = plsc.bitcast(yi, jnp.float32)
    for _ in range(3):
        y = y * (1.5 - 0.5 * v * y * y)
    return y


def _ln_block(rin, rout, g_vec, b_vec):
    it = jax.lax.iota(jnp.int32, L)
    for g in range(BLK // L):
        row_ids = it + (g * L)
        s = jnp.zeros((L,), jnp.float32)
        q = jnp.zeros((L,), jnp.float32)
        cols = []
        for c in range(C):
            cid = jnp.full((L,), c, jnp.int32)
            xc = plsc.load_gather(rin, [row_ids, cid])
            cols.append(xc)
            s = s + xc
            q = q + xc * xc
        mean = s * (1.0 / C)
        var = q * (1.0 / C) - mean * mean
        rstd = _rsqrt(jnp.maximum(var, 0.0) + EPS)
        for c in range(C):
            cid = jnp.full((L,), c, jnp.int32)
            t = (cols[c] - mean) * rstd
            t = t * g_vec[c // L][c % L] + b_vec[c // L][c % L]
            plsc.store_scatter(rout, [row_ids, cid], t)


def _make_sc_kernel(n):
    assert n % (NW * BLK) == 0
    blocks_per_w = n // (NW * BLK)
    assert blocks_per_w % 2 == 0 and blocks_per_w >= 4
    mesh = plsc.VectorSubcoreMesh(core_axis_name="c", subcore_axis_name="s")

    @functools.partial(
        pl.kernel,
        out_type=jax.ShapeDtypeStruct((n, C), jnp.float32),
        mesh=mesh,
        compiler_params=pltpu.CompilerParams(
            needs_layout_passes=False, use_tc_tiling_on_sc=False),
        scratch_types=[
            pltpu.VMEM((2, BLK), jnp.int32),
            pltpu.VMEM((2, BLK, C), jnp.float32),
            pltpu.VMEM((2, BLK, C), jnp.float32),
            pltpu.VMEM((C,), jnp.float32),
            pltpu.VMEM((C,), jnp.float32),
            pltpu.SemaphoreType.DMA,
            pltpu.SemaphoreType.DMA,
            pltpu.SemaphoreType.DMA,
            pltpu.SemaphoreType.DMA,
            pltpu.SemaphoreType.DMA,
            pltpu.SemaphoreType.DMA,
        ],
    )
    def sc_kernel(x_hbm, table_hbm, gamma_hbm, beta_hbm, out_hbm,
                  idx_v, rin_v, rout_v, g_v, b_v,
                  gsem0, gsem1, isem0, isem1, wsem0, wsem1):
        gsem = (gsem0, gsem1)
        isem = (isem0, isem1)
        wsem = (wsem0, wsem1)
        wid = lax.axis_index("s") * NC + lax.axis_index("c")
        base = wid * (blocks_per_w * BLK)
        pltpu.sync_copy(gamma_hbm, g_v)
        pltpu.sync_copy(beta_hbm, b_v)
        g_vec = [g_v[pl.ds(0, L)], g_v[pl.ds(L, L)]]
        b_vec = [b_v[pl.ds(0, L)], b_v[pl.ds(L, L)]]

        for p in range(2):
            pltpu.sync_copy(x_hbm.at[pl.ds(base + p * BLK, BLK)],
                            idx_v.at[p])
            pltpu.async_copy(table_hbm.at[idx_v.at[p]], rin_v.at[p],
                             gsem[p])

        def body(i, carry):
            row0 = base + i * BLK
            for p in range(2):

                @pl.when(i % 2 == p)
                def _():
                    pltpu.make_async_copy(
                        table_hbm.at[idx_v.at[p]], rin_v.at[p],
                        gsem[p]).wait()

                    @pl.when(i + 2 < blocks_per_w)
                    def _():
                        pltpu.async_copy(
                            x_hbm.at[pl.ds(row0 + 2 * BLK, BLK)],
                            idx_v.at[p], isem[p])

                    @pl.when(i >= 2)
                    def _():
                        pltpu.make_async_copy(
                            rout_v.at[p],
                            out_hbm.at[pl.ds(row0 - 2 * BLK, BLK)],
                            wsem[p]).wait()

                    _ln_block(rin_v.at[p], rout_v.at[p], g_vec, b_vec)
                    pltpu.async_copy(rout_v.at[p],
                                     out_hbm.at[pl.ds(row0, BLK)], wsem[p])

                    @pl.when(i + 2 < blocks_per_w)
                    def _():
                        pltpu.make_async_copy(
                            x_hbm.at[pl.ds(row0 + 2 * BLK, BLK)],
                            idx_v.at[p], isem[p]).wait()
                        pltpu.async_copy(table_hbm.at[idx_v.at[p]],
                                         rin_v.at[p], gsem[p])

            return carry

        lax.fori_loop(0, blocks_per_w, body, 0)

        for p in range(2):
            pltpu.make_async_copy(
                rout_v.at[p],
                out_hbm.at[pl.ds(base + (blocks_per_w - 2 + p) * BLK, BLK)],
                wsem[p]).wait()

    return sc_kernel


def kernel(x, table, gamma, beta):
    b, l = x.shape
    n = b * l
    xf = x.reshape(n).astype(jnp.int32)
    out = _make_sc_kernel(n)(xf, table, gamma, beta)
    return out.reshape(b, l, C)


# speedup vs baseline: 2.7159x; 1.0023x over previous
import functools

import jax
import jax.numpy as jnp
from jax import lax
from jax.experimental import pallas as pl
from jax.experimental.pallas import tpu as pltpu
from jax.experimental.pallas import tpu_sc as plsc

C = 32
L = 16
NC, NS = 2, 16
NW = NC * NS
GBLK = 128
BLK = 256
EPS = 1e-5


def _rsqrt(v):
    xi = plsc.bitcast(v, jnp.int32)
    yi = jnp.int32(0x5F3759DF) - (xi >> 1)
    y = plsc.bitcast(yi, jnp.float32)
    for _ in range(2):
        y = y * (1.5 - 0.5 * v * y * y)
    return y


def _ln_half(rin, rout, g_vec, b_vec, half):
    it = jax.lax.iota(jnp.int32, L)
    for g in range(GBLK // L):
        row_ids = it + (half * GBLK + g * L)
        s = jnp.zeros((L,), jnp.float32)
        q = jnp.zeros((L,), jnp.float32)
        cols = []
        for c in range(C):
            cid = jnp.full((L,), c, jnp.int32)
            xc = plsc.load_gather(rin, [row_ids, cid])
            cols.append(xc)
            s = s + xc
            q = q + xc * xc
        mean = s * (1.0 / C)
        var = q * (1.0 / C) - mean * mean
        rstd = _rsqrt(jnp.maximum(var, 0.0) + EPS)
        m2 = mean * rstd
        for c in range(C):
            cid = jnp.full((L,), c, jnp.int32)
            t = cols[c] * rstd - m2
            t = t * g_vec[c // L][c % L] + b_vec[c // L][c % L]
            plsc.store_scatter(rout, [row_ids, cid], t)


def _make_sc_kernel(n):
    assert n % (NW * BLK) == 0
    blocks_per_w = n // (NW * BLK)
    assert blocks_per_w % 2 == 0 and blocks_per_w >= 4
    mesh = plsc.VectorSubcoreMesh(core_axis_name="c", subcore_axis_name="s")

    @functools.partial(
        pl.kernel,
        out_type=jax.ShapeDtypeStruct((n, C), jnp.float32),
        mesh=mesh,
        compiler_params=pltpu.CompilerParams(
            needs_layout_passes=False, use_tc_tiling_on_sc=False),
        scratch_types=[
            pltpu.VMEM((2, 2, GBLK), jnp.int32),
            pltpu.VMEM((2, BLK, C), jnp.float32),
            pltpu.VMEM((2, BLK, C), jnp.float32),
            pltpu.VMEM((C,), jnp.float32),
            pltpu.VMEM((C,), jnp.float32),
            pltpu.SemaphoreType.DMA,
            pltpu.SemaphoreType.DMA,
            pltpu.SemaphoreType.DMA,
            pltpu.SemaphoreType.DMA,
            pltpu.SemaphoreType.DMA,
            pltpu.SemaphoreType.DMA,
            pltpu.SemaphoreType.DMA,
            pltpu.SemaphoreType.DMA,
        ],
    )
    def sc_kernel(x_hbm, table_hbm, gamma_hbm, beta_hbm, out_hbm,
                  idx_v, rin_v, rout_v, g_v, b_v,
                  gsem00, gsem01, gsem10, gsem11, isem0, isem1,
                  wsem0, wsem1):
        gsem = ((gsem00, gsem01), (gsem10, gsem11))
        isem = (isem0, isem1)
        wsem = (wsem0, wsem1)
        wid = lax.axis_index("s") * NC + lax.axis_index("c")
        base = wid * (blocks_per_w * BLK)
        pltpu.sync_copy(gamma_hbm, g_v)
        pltpu.sync_copy(beta_hbm, b_v)
        g_vec = [g_v[pl.ds(0, L)], g_v[pl.ds(L, L)]]
        b_vec = [b_v[pl.ds(0, L)], b_v[pl.ds(L, L)]]

        def launch_gathers(p):
            for h in range(2):
                pltpu.async_copy(
                    table_hbm.at[idx_v.at[p, h]],
                    rin_v.at[p, pl.ds(h * GBLK, GBLK)], gsem[p][h])

        def wait_gather(p, h):
            pltpu.make_async_copy(
                table_hbm.at[idx_v.at[p, h]],
                rin_v.at[p, pl.ds(h * GBLK, GBLK)], gsem[p][h]).wait()

        for p in range(2):
            for h in range(2):
                pltpu.sync_copy(
                    x_hbm.at[pl.ds(base + (p * 2 + h) * GBLK, GBLK)],
                    idx_v.at[p, h])
            launch_gathers(p)

        def body(i, carry):
            row0 = base + i * BLK
            for p in range(2):

                @pl.when(i % 2 == p)
                def _():
                    wait_gather(p, 0)

                    @pl.when(i + 2 < blocks_per_w)
                    def _():
                        for h in range(2):
                            pltpu.async_copy(
                                x_hbm.at[pl.ds(row0 + 2 * BLK + h * GBLK,
                                               GBLK)],
                                idx_v.at[p, h], isem[p])

                    @pl.when(i >= 2)
                    def _():
                        pltpu.make_async_copy(
                            rout_v.at[p],
                            out_hbm.at[pl.ds(row0 - 2 * BLK, BLK)],
                            wsem[p]).wait()

                    _ln_half(rin_v.at[p], rout_v.at[p], g_vec, b_vec, 0)
                    wait_gather(p, 1)
                    _ln_half(rin_v.at[p], rout_v.at[p], g_vec, b_vec, 1)
                    pltpu.async_copy(rout_v.at[p],
                                     out_hbm.at[pl.ds(row0, BLK)], wsem[p])

                    @pl.when(i + 2 < blocks_per_w)
                    def _():
                        for h in range(2):
                            pltpu.make_async_copy(
                                x_hbm.at[pl.ds(row0 + 2 * BLK + h * GBLK,
                                               GBLK)],
                                idx_v.at[p, h], isem[p]).wait()
                        launch_gathers(p)

            return carry

        lax.fori_loop(0, blocks_per_w, body, 0)

        for p in range(2):
            pltpu.make_async_copy(
                rout_v.at[p],
                out_hbm.at[pl.ds(base + (blocks_per_w - 2 + p) * BLK, BLK)],
                wsem[p]).wait()

    return sc_kernel


def kernel(x, table, gamma, beta):
    b, l = x.shape
    n = b * l
    xf = x.reshape(n).astype(jnp.int32)
    out = _make_sc_kernel(n)(xf, table, gamma, beta)
    return out.reshape(b, l, C)


# speedup vs baseline: 3.6099x; 1.3292x over previous
import functools

import jax
import jax.numpy as jnp
from jax import lax
from jax.experimental import pallas as pl
from jax.experimental.pallas import tpu as pltpu
from jax.experimental.pallas import tpu_sc as plsc

C = 32
L = 16
NC, NS = 2, 16
NW = NC * NS
GBLK = 128
BLK = 256
EPS = 1e-5


def _rsqrt(v):
    xi = plsc.bitcast(v, jnp.int32)
    yi = jnp.int32(0x5F3759DF) - (xi >> 1)
    y = plsc.bitcast(yi, jnp.float32)
    for _ in range(2):
        y = y * (1.5 - 0.5 * v * y * y)
    return y


def _ln_half(rin, rout, g_vec, b_vec, half):
    it = jax.lax.iota(jnp.int32, L)

    def group(g, carry):
        base_r = half * GBLK + g * L
        row_ids = it + base_r
        s = jnp.zeros((L,), jnp.float32)
        q = jnp.zeros((L,), jnp.float32)
        for c in range(C):
            cid = (it + c) & (C - 1)
            xc = plsc.load_gather(rin, [row_ids, cid])
            s = s + xc
            q = q + xc * xc
        mean = s * (1.0 / C)
        var = q * (1.0 / C) - mean * mean
        rstd = _rsqrt(jnp.maximum(var, 0.0) + EPS)
        m2 = mean * rstd
        for r in range(L):
            rs = rstd[r]
            ms = m2[r]
            for hc in range(2):
                a = rin[base_r + r, pl.ds(hc * L, L)]
                y = a * rs - ms
                y = y * g_vec[hc] + b_vec[hc]
                rout[base_r + r, pl.ds(hc * L, L)] = y
        return carry

    lax.fori_loop(0, GBLK // L, group, 0)


def _make_sc_kernel(n):
    assert n % (NW * BLK) == 0
    blocks_per_w = n // (NW * BLK)
    assert blocks_per_w % 2 == 0 and blocks_per_w >= 4
    mesh = plsc.VectorSubcoreMesh(core_axis_name="c", subcore_axis_name="s")

    @functools.partial(
        pl.kernel,
        out_type=jax.ShapeDtypeStruct((n, C), jnp.float32),
        mesh=mesh,
        compiler_params=pltpu.CompilerParams(
            needs_layout_passes=False, use_tc_tiling_on_sc=False),
        scratch_types=[
            pltpu.VMEM((2, 2, GBLK), jnp.int32),
            pltpu.VMEM((2, BLK, C), jnp.float32),
            pltpu.VMEM((2, BLK, C), jnp.float32),
            pltpu.VMEM((C,), jnp.float32),
            pltpu.VMEM((C,), jnp.float32),
            pltpu.SemaphoreType.DMA,
            pltpu.SemaphoreType.DMA,
            pltpu.SemaphoreType.DMA,
            pltpu.SemaphoreType.DMA,
            pltpu.SemaphoreType.DMA,
            pltpu.SemaphoreType.DMA,
            pltpu.SemaphoreType.DMA,
            pltpu.SemaphoreType.DMA,
        ],
    )
    def sc_kernel(x_hbm, table_hbm, gamma_hbm, beta_hbm, out_hbm,
                  idx_v, rin_v, rout_v, g_v, b_v,
                  gsem00, gsem01, gsem10, gsem11, isem0, isem1,
                  wsem0, wsem1):
        gsem = ((gsem00, gsem01), (gsem10, gsem11))
        isem = (isem0, isem1)
        wsem = (wsem0, wsem1)
        wid = lax.axis_index("s") * NC + lax.axis_index("c")
        base = wid * (blocks_per_w * BLK)
        pltpu.sync_copy(gamma_hbm, g_v)
        pltpu.sync_copy(beta_hbm, b_v)
        g_vec = [g_v[pl.ds(0, L)], g_v[pl.ds(L, L)]]
        b_vec = [b_v[pl.ds(0, L)], b_v[pl.ds(L, L)]]

        def launch_gathers(p):
            for h in range(2):
                pltpu.async_copy(
                    table_hbm.at[idx_v.at[p, h]],
                    rin_v.at[p, pl.ds(h * GBLK, GBLK)], gsem[p][h])

        def wait_gather(p, h):
            pltpu.make_async_copy(
                table_hbm.at[idx_v.at[p, h]],
                rin_v.at[p, pl.ds(h * GBLK, GBLK)], gsem[p][h]).wait()

        for p in range(2):
            for h in range(2):
                pltpu.sync_copy(
                    x_hbm.at[pl.ds(base + (p * 2 + h) * GBLK, GBLK)],
                    idx_v.at[p, h])
            launch_gathers(p)

        def body(i, carry):
            row0 = base + i * BLK
            for p in range(2):

                @pl.when(i % 2 == p)
                def _():
                    wait_gather(p, 0)

                    @pl.when(i + 2 < blocks_per_w)
                    def _():
                        for h in range(2):
                            pltpu.async_copy(
                                x_hbm.at[pl.ds(row0 + 2 * BLK + h * GBLK,
                                               GBLK)],
                                idx_v.at[p, h], isem[p])

                    @pl.when(i >= 2)
                    def _():
                        pltpu.make_async_copy(
                            rout_v.at[p],
                            out_hbm.at[pl.ds(row0 - 2 * BLK, BLK)],
                            wsem[p]).wait()

                    _ln_half(rin_v.at[p], rout_v.at[p], g_vec, b_vec, 0)
                    wait_gather(p, 1)
                    _ln_half(rin_v.at[p], rout_v.at[p], g_vec, b_vec, 1)
                    pltpu.async_copy(rout_v.at[p],
                                     out_hbm.at[pl.ds(row0, BLK)], wsem[p])

                    @pl.when(i + 2 < blocks_per_w)
                    def _():
                        for h in range(2):
                            pltpu.make_async_copy(
                                x_hbm.at[pl.ds(row0 + 2 * BLK + h * GBLK,
                                               GBLK)],
                                idx_v.at[p, h], isem[p]).wait()
                        launch_gathers(p)

            return carry

        lax.fori_loop(0, blocks_per_w, body, 0)

        for p in range(2):
            pltpu.make_async_copy(
                rout_v.at[p],
                out_hbm.at[pl.ds(base + (blocks_per_w - 2 + p) * BLK, BLK)],
                wsem[p]).wait()

    return sc_kernel


def kernel(x, table, gamma, beta):
    b, l = x.shape
    n = b * l
    xf = x.reshape(n).astype(jnp.int32)
    out = _make_sc_kernel(n)(xf, table, gamma, beta)
    return out.reshape(b, l, C)


# speedup vs baseline: 6.2603x; 1.7342x over previous
import functools

import jax
import jax.numpy as jnp
from jax import lax
from jax.experimental import pallas as pl
from jax.experimental.pallas import tpu as pltpu
from jax.experimental.pallas import tpu_sc as plsc

C = 32
NC, NS = 2, 16
NW = NC * NS
GBLK = 128
BLK = 256
P = 4
EPS = 1e-5
TC_BLK = 1000


def _tc_ln_body(t_ref, g_ref, b_ref, o_ref):
    x = t_ref[...]
    mean = jnp.mean(x, axis=1, keepdims=True)
    var = jnp.mean((x - mean) ** 2, axis=1, keepdims=True)
    normed = (x - mean) / jnp.sqrt(var + EPS)
    o_ref[...] = normed * g_ref[...] + b_ref[...]


def _normalize_table(table, gamma, beta):
    v = table.shape[0]
    assert v % TC_BLK == 0
    return pl.pallas_call(
        _tc_ln_body,
        grid=(v // TC_BLK,),
        in_specs=[
            pl.BlockSpec((TC_BLK, C), lambda i: (i, 0)),
            pl.BlockSpec((1, C), lambda i: (0, 0)),
            pl.BlockSpec((1, C), lambda i: (0, 0)),
        ],
        out_specs=pl.BlockSpec((TC_BLK, C), lambda i: (i, 0)),
        out_shape=jax.ShapeDtypeStruct((v, C), jnp.float32),
    )(table, gamma.reshape(1, C), beta.reshape(1, C))


def _make_sc_gather(n):
    assert n % (NW * BLK) == 0
    blocks_per_w = n // (NW * BLK)
    assert blocks_per_w >= P
    mesh = plsc.VectorSubcoreMesh(core_axis_name="c", subcore_axis_name="s")

    @functools.partial(
        pl.kernel,
        out_type=jax.ShapeDtypeStruct((n, C), jnp.float32),
        mesh=mesh,
        compiler_params=pltpu.CompilerParams(
            needs_layout_passes=False, use_tc_tiling_on_sc=False),
        scratch_types=[
            pltpu.VMEM((P, 2, GBLK), jnp.int32),
            pltpu.VMEM((P, BLK, C), jnp.float32),
        ] + [pltpu.SemaphoreType.DMA] * (3 * P),
    )
    def sc_gather(x_hbm, table_hbm, out_hbm, idx_v, rows_v, *sems):
        gsem = sems[0:P]
        isem = sems[P:2 * P]
        wsem = sems[2 * P:3 * P]
        wid = lax.axis_index("s") * NC + lax.axis_index("c")
        base = wid * (blocks_per_w * BLK)

        def launch_gathers(p):
            for h in range(2):
                pltpu.async_copy(
                    table_hbm.at[idx_v.at[p, h]],
                    rows_v.at[p, pl.ds(h * GBLK, GBLK)], gsem[p])

        def wait_gathers(p):
            for h in range(2):
                pltpu.make_async_copy(
                    table_hbm.at[idx_v.at[p, h]],
                    rows_v.at[p, pl.ds(h * GBLK, GBLK)], gsem[p]).wait()

        for p in range(2):
            for h in range(2):
                pltpu.sync_copy(
                    x_hbm.at[pl.ds(base + (p * 2 + h) * GBLK, GBLK)],
                    idx_v.at[p, h])
            launch_gathers(p)

        def body(i, carry):
            row0 = base + i * BLK
            for p in range(P):

                @pl.when(i % P == p)
                def _():
                    p2 = (p + 2) % P

                    @pl.when(i + 2 < blocks_per_w)
                    def _():
                        for h in range(2):
                            pltpu.async_copy(
                                x_hbm.at[pl.ds(row0 + 2 * BLK + h * GBLK,
                                               GBLK)],
                                idx_v.at[p2, h], isem[p2])

                    wait_gathers(p)
                    pltpu.async_copy(rows_v.at[p],
                                     out_hbm.at[pl.ds(row0, BLK)], wsem[p])

                    @pl.when(i + 2 < blocks_per_w)
                    def _():
                        @pl.when(i >= 2)
                        def _():
                            pltpu.make_async_copy(
                                rows_v.at[p2],
                                out_hbm.at[pl.ds(row0 - 2 * BLK, BLK)],
                                wsem[p2]).wait()
                        for h in range(2):
                            pltpu.make_async_copy(
                                x_hbm.at[pl.ds(row0 + 2 * BLK + h * GBLK,
                                               GBLK)],
                                idx_v.at[p2, h], isem[p2]).wait()
                        launch_gathers(p2)

            return carry

        lax.fori_loop(0, blocks_per_w, body, 0)

        for k in range(4):
            i_last = blocks_per_w - 1 - k
            pltpu.make_async_copy(
                rows_v.at[i_last % P],
                out_hbm.at[pl.ds(base + i_last * BLK, BLK)],
                wsem[i_last % P]).wait()

    return sc_gather


def kernel(x, table, gamma, beta):
    b, l = x.shape
    n = b * l
    xf = x.reshape(n).astype(jnp.int32)
    table_n = _normalize_table(table, gamma, beta)
    out = _make_sc_gather(n)(xf, table_n)
    return out.reshape(b, l, C)
